# baseline (speedup 1.0000x reference)
import functools

import jax
import jax.numpy as jnp
import numpy as np
from jax import lax
from jax.experimental import pallas as pl
from jax.experimental.pallas import tpu as pltpu
from jax.experimental.pallas import tpu_sc as plsc

NC, NS = 2, 16
NW = NC * NS
BATCH, SEQ = 4, 2048
N_IDX = BATCH * SEQ
D = 1024
ROWS_PER_W = N_IDX // NW
W_PER_ROW = SEQ // ROWS_PER_W
CH = 16
NCHUNK = ROWS_PER_W // CH
NBUF = 4
NFBUF = 2
L = 16

_mesh = plsc.VectorSubcoreMesh(core_axis_name="c", subcore_axis_name="s")


@functools.partial(
    pl.kernel,
    mesh=_mesh,
    compiler_params=pltpu.CompilerParams(needs_layout_passes=False),
    out_type=jax.ShapeDtypeStruct((N_IDX * D,), jnp.float32),
    scratch_types=[
        pltpu.VMEM((ROWS_PER_W,), jnp.int32),
    ]
    + [pltpu.VMEM((CH, D // 2), jnp.int32) for _ in range(NBUF)]
    + [pltpu.VMEM((CH * D,), jnp.float32) for _ in range(NFBUF)]
    + [
        pltpu.SemaphoreType.DMA,
        pltpu.SemaphoreType.DMA,
    ],
)
def _gather_kernel(x_hbm, pe_hbm, out_hbm, idx_v, *rest):
    bbufs = rest[:NBUF]
    fbufs = rest[NBUF:NBUF + NFBUF]
    gsem, osem = rest[NBUF + NFBUF], rest[NBUF + NFBUF + 1]
    wid = lax.axis_index("s") * NC + lax.axis_index("c")
    base = wid * ROWS_PER_W

    pltpu.sync_copy(
        x_hbm.at[wid // W_PER_ROW, pl.ds((wid % W_PER_ROW) * ROWS_PER_W, ROWS_PER_W)],
        idx_v,
    )

    def gather(c, bbuf):
        pltpu.async_copy(pe_hbm.at[idx_v.at[pl.ds(c * CH, CH)]], bbuf, gsem)

    himask = jnp.full((L,), np.int32(-65536), jnp.int32)

    def widen(bbuf, fbuf):
        @pl.loop(0, CH)
        def _rows(r):
            rbase = r * D
            for k in range(D // 32):
                w = bbuf[r, pl.ds(k * L, L)]
                lo = plsc.bitcast(lax.shift_left(w, 16), jnp.float32)
                hi = plsc.bitcast(lax.bitwise_and(w, himask), jnp.float32)
                cbase = rbase + k * 32
                fbuf[pl.ds(cbase, L)] = lo
                fbuf[pl.ds(cbase + L, L)] = hi

    for b in range(NBUF):
        gather(b, bbufs[b])

    @pl.loop(0, NCHUNK, step=NBUF)
    def _chunks(i):
        for b in range(NBUF):
            c = i + b
            fb = b % NFBUF
            pltpu.make_async_copy(
                pe_hbm.at[pl.ds(0, CH)], bbufs[b], gsem
            ).wait()

            @pl.when(c >= NFBUF)
            def _():
                pltpu.make_async_copy(
                    fbufs[fb], out_hbm.at[pl.ds(0, CH * D)], osem
                ).wait()

            widen(bbufs[b], fbufs[fb])
            pltpu.async_copy(
                fbufs[fb],
                out_hbm.at[pl.ds((base + c * CH) * D, CH * D)],
                osem,
            )

            @pl.when(c + NBUF < NCHUNK)
            def _():
                gather(c + NBUF, bbufs[b])

    for fb in range(NFBUF):
        pltpu.make_async_copy(
            fbufs[fb], out_hbm.at[pl.ds(0, CH * D)], osem
        ).wait()


_grp = np.arange(D, dtype=np.int32).reshape(D // 32, 32)
_perm = np.empty_like(_grp)
_perm[:, 0::2] = _grp[:, :16]
_perm[:, 1::2] = _grp[:, 16:]
_PERM = jnp.asarray(_perm.reshape(-1))


def kernel(x, pe):
    pe_w = lax.bitcast_convert_type(
        pe[:, _PERM].astype(jnp.bfloat16).reshape(4096, D // 2, 2), jnp.int32
    )
    out = _gather_kernel(x, pe_w)
    return out.reshape(BATCH, SEQ, D)


# speedup vs baseline: 1.0887x; 1.0887x over previous
import functools

import jax
import jax.numpy as jnp
import numpy as np
from jax import lax
from jax.experimental import pallas as pl
from jax.experimental.pallas import tpu as pltpu
from jax.experimental.pallas import tpu_sc as plsc

NC, NS = 2, 16
NW = NC * NS
BATCH, SEQ = 4, 2048
N_IDX = BATCH * SEQ
D = 1024
ROWS_PER_W = N_IDX // NW
W_PER_ROW = SEQ // ROWS_PER_W
CH = 16
NCHUNK = ROWS_PER_W // CH
NBUF = 4
NFBUF = 2
L = 16

_mesh = plsc.VectorSubcoreMesh(core_axis_name="c", subcore_axis_name="s")


@functools.partial(
    pl.kernel,
    mesh=_mesh,
    compiler_params=pltpu.CompilerParams(needs_layout_passes=False),
    out_type=jax.ShapeDtypeStruct((N_IDX * D,), jnp.float32),
    scratch_types=[
        pltpu.VMEM((ROWS_PER_W,), jnp.int32),
    ]
    + [pltpu.VMEM((CH, D // 2), jnp.int32) for _ in range(NBUF)]
    + [pltpu.VMEM((CH * D,), jnp.float32) for _ in range(NFBUF)]
    + [
        pltpu.SemaphoreType.DMA,
        pltpu.SemaphoreType.DMA,
    ],
)
def _gather_kernel(x_hbm, pe_hbm, out_hbm, idx_v, *rest):
    bbufs = rest[:NBUF]
    fbufs = rest[NBUF:NBUF + NFBUF]
    gsem, osem = rest[NBUF + NFBUF], rest[NBUF + NFBUF + 1]
    wid = lax.axis_index("s") * NC + lax.axis_index("c")
    base = wid * ROWS_PER_W

    pltpu.sync_copy(
        x_hbm.at[wid // W_PER_ROW, pl.ds((wid % W_PER_ROW) * ROWS_PER_W, ROWS_PER_W)],
        idx_v,
    )

    def gather(c, bbuf):
        pltpu.async_copy(pe_hbm.at[idx_v.at[pl.ds(c * CH, CH)]], bbuf, gsem)

    himask = jnp.full((L,), np.int32(-65536), jnp.int32)

    def widen(bbuf, fbuf):
        @plsc.parallel_loop(0, CH, unroll=2)
        def _rows(r):
            rbase = r * D
            for k in range(D // 32):
                w = bbuf[r, pl.ds(k * L, L)]
                lo = plsc.bitcast(lax.shift_left(w, 16), jnp.float32)
                hi = plsc.bitcast(lax.bitwise_and(w, himask), jnp.float32)
                cbase = rbase + k * 32
                fbuf[pl.ds(cbase, L)] = lo
                fbuf[pl.ds(cbase + L, L)] = hi

    for b in range(NBUF):
        gather(b, bbufs[b])

    @pl.loop(0, NCHUNK, step=NBUF)
    def _chunks(i):
        for b in range(NBUF):
            c = i + b
            fb = b % NFBUF
            pltpu.make_async_copy(
                pe_hbm.at[pl.ds(0, CH)], bbufs[b], gsem
            ).wait()

            @pl.when(c >= NFBUF)
            def _():
                pltpu.make_async_copy(
                    fbufs[fb], out_hbm.at[pl.ds(0, CH * D)], osem
                ).wait()

            widen(bbufs[b], fbufs[fb])
            pltpu.async_copy(
                fbufs[fb],
                out_hbm.at[pl.ds((base + c * CH) * D, CH * D)],
                osem,
            )

            @pl.when(c + NBUF < NCHUNK)
            def _():
                gather(c + NBUF, bbufs[b])

    for fb in range(NFBUF):
        pltpu.make_async_copy(
            fbufs[fb], out_hbm.at[pl.ds(0, CH * D)], osem
        ).wait()


_grp = np.arange(D, dtype=np.int32).reshape(D // 32, 32)
_perm = np.empty_like(_grp)
_perm[:, 0::2] = _grp[:, :16]
_perm[:, 1::2] = _grp[:, 16:]
_PERM = jnp.asarray(_perm.reshape(-1))


def kernel(x, pe):
    pe_w = lax.bitcast_convert_type(
        pe[:, _PERM].astype(jnp.bfloat16).reshape(4096, D // 2, 2), jnp.int32
    )
    out = _gather_kernel(x, pe_w)
    return out.reshape(BATCH, SEQ, D)


# speedup vs baseline: 4.6639x; 4.2838x over previous
import functools

import jax
import jax.numpy as jnp
from jax import lax
from jax.experimental import pallas as pl
from jax.experimental.pallas import tpu as pltpu
from jax.experimental.pallas import tpu_sc as plsc

NC, NS = 2, 16
NW = NC * NS
BATCH, SEQ = 4, 2048
N_IDX = BATCH * SEQ
D = 1024
ROWS_PER_W = N_IDX // NW
W_PER_ROW = SEQ // ROWS_PER_W
CH = 16
NCHUNK = ROWS_PER_W // CH
NBUF = 4

_mesh = plsc.VectorSubcoreMesh(core_axis_name="c", subcore_axis_name="s")


@functools.partial(
    pl.kernel,
    mesh=_mesh,
    out_type=jax.ShapeDtypeStruct((N_IDX, D), jnp.float32),
    scratch_types=[
        pltpu.VMEM((ROWS_PER_W,), jnp.int32),
    ]
    + [pltpu.VMEM((CH, D), jnp.float32) for _ in range(NBUF)]
    + [
        pltpu.SemaphoreType.DMA,
        pltpu.SemaphoreType.DMA,
    ],
)
def _gather_kernel(x_hbm, pe_hbm, out_hbm, idx_v, *rest):
    bufs = rest[:NBUF]
    gsem, osem = rest[NBUF], rest[NBUF + 1]
    wid = lax.axis_index("s") * NC + lax.axis_index("c")
    base = wid * ROWS_PER_W

    pltpu.sync_copy(
        x_hbm.at[wid // W_PER_ROW, pl.ds((wid % W_PER_ROW) * ROWS_PER_W, ROWS_PER_W)],
        idx_v,
    )

    def gather(c, buf):
        pltpu.async_copy(pe_hbm.at[idx_v.at[pl.ds(c * CH, CH)]], buf, gsem)

    for b in range(NBUF):
        gather(b, bufs[b])

    @pl.loop(0, NCHUNK, step=NBUF)
    def _chunks(i):
        for b in range(NBUF):
            c = i + b
            pltpu.make_async_copy(pe_hbm.at[pl.ds(0, CH)], bufs[b], gsem).wait()
            out_cp = pltpu.async_copy(
                bufs[b], out_hbm.at[pl.ds(base + c * CH, CH)], osem
            )
            out_cp.wait()

            @pl.when(c + NBUF < NCHUNK)
            def _():
                gather(c + NBUF, bufs[b])


def kernel(x, pe):
    out = _gather_kernel(x, pe)
    return out.reshape(BATCH, SEQ, D)
